# baseline (speedup 1.0000x reference)
import functools

import jax
import jax.numpy as jnp
from jax import lax
from jax.experimental import pallas as pl
from jax.experimental.pallas import tpu as pltpu
from jax.experimental.pallas import tpu_sc as plsc

_EMB = 128
_NFV = _EMB // 16
_NBUF = 4

_GATHER_DNUMS = lax.GatherDimensionNumbers(
    offset_dims=(), collapsed_slice_dims=(0,), start_index_map=(0,))


def _lane_shuffle(x, idx):
    return lax.gather(x, idx[:, None], _GATHER_DNUMS, slice_sizes=(1,),
                      mode=lax.GatherScatterMode.PROMISE_IN_BOUNDS)


def _lane_sum(x, lanes):
    for sh in (8, 4, 2, 1):
        x = x + _lane_shuffle(x, jnp.bitwise_xor(lanes, sh))
    return x


def _sc_body(rows_per, C, h_hbm, t_hbm, r_hbm, ent_hbm, relnorm_hbm,
             out_hbm, h_idx, t_idx, r_idx, h_rows, t_rows, rn_rows,
             out_v, sem):
    nc = 2
    sid = lax.axis_index("s")
    wid = sid * nc + lax.axis_index("c")
    lanes = lax.iota(jnp.int32, 16)
    nchunks = rows_per // C
    wbase = wid * rows_per

    pltpu.sync_copy(h_hbm.at[pl.ds(wbase, rows_per)], h_idx)
    pltpu.sync_copy(t_hbm.at[pl.ds(wbase, rows_per)], t_idx)
    pltpu.sync_copy(r_hbm.at[pl.ds(wbase, rows_per)], r_idx)

    def copies(c, b):
        off = c * C
        return (
            pltpu.make_async_copy(ent_hbm.at[h_idx.at[pl.ds(off, C)]],
                                  h_rows.at[b], sem.at[b]),
            pltpu.make_async_copy(ent_hbm.at[t_idx.at[pl.ds(off, C)]],
                                  t_rows.at[b], sem.at[b]),
            pltpu.make_async_copy(relnorm_hbm.at[r_idx.at[pl.ds(off, C)]],
                                  rn_rows.at[b], sem.at[b]),
        )

    def fire(c, b):
        for cp in copies(c, b):
            cp.start()

    def drain(c, b):
        for cp in copies(c, b):
            cp.wait()

    def compute(c, b):
        def score_row(i):
            def load2(ref, col16):
                w = ref[b, i, pl.ds(col16 * 16, 16)]
                lo = lax.bitcast_convert_type(
                    lax.shift_left(w, 16), jnp.float32)
                hi = lax.bitcast_convert_type(
                    jnp.bitwise_and(w, jnp.int32(-65536)), jnp.float32)
                return lo, hi

            dk = []
            nk = []
            accd = jnp.zeros((16,), jnp.float32)
            for k in range(_NFV // 2):
                n2 = load2(rn_rows, _NFV // 2 + k)
                for u in range(2):
                    kk = 2 * k + u
                    hv = h_rows[b, i, pl.ds(kk * 16, 16)]
                    tv = t_rows[b, i, pl.ds(kk * 16, 16)]
                    d = hv - tv
                    dk.append(d)
                    nk.append(n2[u])
                    accd = accd + d * n2[u]
            s = _lane_sum(accd, lanes)
            acc2 = jnp.zeros((16,), jnp.float32)
            for k in range(_NFV // 2):
                r2 = load2(rn_rows, k)
                for u in range(2):
                    e = dk[2 * k + u] + r2[u] - s * nk[2 * k + u]
                    acc2 = acc2 + jnp.abs(e)
            return _lane_sum(acc2, lanes)

        def group_body(g, _):
            def row_body(q, res):
                sc0 = score_row(g * 16 + q)
                return jnp.where(lanes == q, sc0, res)

            res = lax.fori_loop(0, 16, row_body,
                                jnp.zeros((16,), jnp.float32))
            out_v[pl.ds(c * C + g * 16, 16)] = res
            return 0

        lax.fori_loop(0, C // 16, group_body, 0)

    for b0 in range(_NBUF - 1):
        fire(b0, b0)

    def ring_body(c):
        b = lax.rem(c, _NBUF)

        @pl.when(c + _NBUF - 1 < nchunks)
        def _():
            fire(c + _NBUF - 1, lax.rem(c + _NBUF - 1, _NBUF))

        drain(c, b)
        compute(c, b)

    pl.loop(0, nchunks)(ring_body)
    pltpu.sync_copy(out_v, out_hbm.at[pl.ds(wbase, rows_per)])


def kernel(h, t, r, ent_emb, rel_emb, norm_emb):
    batch = h.shape[0]
    nw = 32
    rows_per = batch // nw
    C = 64
    def to_i32pairs(x):
        n = x.shape[0]
        xb = x.astype(jnp.bfloat16).reshape(n, x.shape[1] // 32, 2, 16)
        xb = jnp.transpose(xb, (0, 1, 3, 2))
        return lax.bitcast_convert_type(xb, jnp.int32).reshape(n, -1)

    relnorm = jnp.concatenate(
        [to_i32pairs(rel_emb), to_i32pairs(norm_emb)], axis=1)
    mesh = plsc.VectorSubcoreMesh(core_axis_name="c", subcore_axis_name="s")
    run = pl.kernel(
        functools.partial(_sc_body, rows_per, C),
        out_type=jax.ShapeDtypeStruct((batch,), jnp.float32),
        mesh=mesh,
        scratch_types=[
            pltpu.VMEM((rows_per,), jnp.int32),
            pltpu.VMEM((rows_per,), jnp.int32),
            pltpu.VMEM((rows_per,), jnp.int32),
            pltpu.VMEM((_NBUF, C, _EMB), jnp.float32),
            pltpu.VMEM((_NBUF, C, _EMB), jnp.float32),
            pltpu.VMEM((_NBUF, C, _EMB), jnp.int32),
            pltpu.VMEM((rows_per,), jnp.float32),
            pltpu.SemaphoreType.DMA((_NBUF,)),
        ],
    )
    return run(h, t, r, ent_emb, relnorm)


# speedup vs baseline: 1.0914x; 1.0914x over previous
import functools

import jax
import jax.numpy as jnp
from jax import lax
from jax.experimental import pallas as pl
from jax.experimental.pallas import tpu as pltpu
from jax.experimental.pallas import tpu_sc as plsc

_EMB = 128
_NFV = _EMB // 16
_NBUF = 4

_GATHER_DNUMS = lax.GatherDimensionNumbers(
    offset_dims=(), collapsed_slice_dims=(0,), start_index_map=(0,))


def _lane_shuffle(x, idx):
    return lax.gather(x, idx[:, None], _GATHER_DNUMS, slice_sizes=(1,),
                      mode=lax.GatherScatterMode.PROMISE_IN_BOUNDS)


def _lane_sum(x, lanes):
    for sh in (8, 4, 2, 1):
        x = x + _lane_shuffle(x, jnp.bitwise_xor(lanes, sh))
    return x


def _sc_body(rows_per, C, h_hbm, t_hbm, r_hbm, ent_hbm, relnorm_hbm,
             out_hbm, h_idx, t_idx, r_idx, h_rows, t_rows, rn_rows,
             out_v, sem):
    nc = 2
    sid = lax.axis_index("s")
    wid = sid * nc + lax.axis_index("c")
    lanes = lax.iota(jnp.int32, 16)
    nchunks = rows_per // C
    wbase = wid * rows_per

    idx_cps = (
        pltpu.make_async_copy(h_hbm.at[pl.ds(wbase, rows_per)], h_idx,
                              sem.at[_NBUF]),
        pltpu.make_async_copy(t_hbm.at[pl.ds(wbase, rows_per)], t_idx,
                              sem.at[_NBUF]),
        pltpu.make_async_copy(r_hbm.at[pl.ds(wbase, rows_per)], r_idx,
                              sem.at[_NBUF]),
    )
    for cp in idx_cps:
        cp.start()
    for cp in idx_cps:
        cp.wait()

    def copies(c, b):
        off = c * C
        return (
            pltpu.make_async_copy(ent_hbm.at[h_idx.at[pl.ds(off, C)]],
                                  h_rows.at[b], sem.at[b]),
            pltpu.make_async_copy(ent_hbm.at[t_idx.at[pl.ds(off, C)]],
                                  t_rows.at[b], sem.at[b]),
            pltpu.make_async_copy(relnorm_hbm.at[r_idx.at[pl.ds(off, C)]],
                                  rn_rows.at[b], sem.at[b]),
        )

    def fire(c, b):
        for cp in copies(c, b):
            cp.start()

    def drain(c, b):
        for cp in copies(c, b):
            cp.wait()

    def compute(c, b):
        def score_row(i):
            def load2(ref, col16):
                w = ref[b, i, pl.ds(col16 * 16, 16)]
                lo = lax.bitcast_convert_type(
                    lax.shift_left(w, 16), jnp.float32)
                hi = lax.bitcast_convert_type(
                    jnp.bitwise_and(w, jnp.int32(-65536)), jnp.float32)
                return lo, hi

            dk = []
            nk = []
            accd = jnp.zeros((16,), jnp.float32)
            for k in range(_NFV // 2):
                n2 = load2(rn_rows, _NFV // 2 + k)
                for u in range(2):
                    kk = 2 * k + u
                    hv = h_rows[b, i, pl.ds(kk * 16, 16)]
                    tv = t_rows[b, i, pl.ds(kk * 16, 16)]
                    d = hv - tv
                    dk.append(d)
                    nk.append(n2[u])
                    accd = accd + d * n2[u]
            s = _lane_sum(accd, lanes)
            acc2 = jnp.zeros((16,), jnp.float32)
            for k in range(_NFV // 2):
                r2 = load2(rn_rows, k)
                for u in range(2):
                    e = dk[2 * k + u] + r2[u] - s * nk[2 * k + u]
                    acc2 = acc2 + jnp.abs(e)
            return _lane_sum(acc2, lanes)

        def group_body(g, _):
            def row_body(q, res):
                sc0 = score_row(g * 16 + q)
                return jnp.where(lanes == q, sc0, res)

            res = lax.fori_loop(0, 16, row_body,
                                jnp.zeros((16,), jnp.float32))
            out_v[pl.ds(c * C + g * 16, 16)] = res
            return 0

        lax.fori_loop(0, C // 16, group_body, 0)

    for b0 in range(_NBUF - 1):
        fire(b0, b0)

    def ring_body(c):
        b = lax.rem(c, _NBUF)

        @pl.when(c + _NBUF - 1 < nchunks)
        def _():
            fire(c + _NBUF - 1, lax.rem(c + _NBUF - 1, _NBUF))

        drain(c, b)
        compute(c, b)

    pl.loop(0, nchunks)(ring_body)
    pltpu.sync_copy(out_v, out_hbm.at[pl.ds(wbase, rows_per)])


def kernel(h, t, r, ent_emb, rel_emb, norm_emb):
    batch = h.shape[0]
    nw = 32
    rows_per = batch // nw
    C = 32
    def to_i32pairs(x):
        n = x.shape[0]
        xb = x.astype(jnp.bfloat16).reshape(n, x.shape[1] // 32, 2, 16)
        xb = jnp.transpose(xb, (0, 1, 3, 2))
        return lax.bitcast_convert_type(xb, jnp.int32).reshape(n, -1)

    relnorm = jnp.concatenate(
        [to_i32pairs(rel_emb), to_i32pairs(norm_emb)], axis=1)
    mesh = plsc.VectorSubcoreMesh(core_axis_name="c", subcore_axis_name="s")
    run = pl.kernel(
        functools.partial(_sc_body, rows_per, C),
        out_type=jax.ShapeDtypeStruct((batch,), jnp.float32),
        mesh=mesh,
        scratch_types=[
            pltpu.VMEM((rows_per,), jnp.int32),
            pltpu.VMEM((rows_per,), jnp.int32),
            pltpu.VMEM((rows_per,), jnp.int32),
            pltpu.VMEM((_NBUF, C, _EMB), jnp.float32),
            pltpu.VMEM((_NBUF, C, _EMB), jnp.float32),
            pltpu.VMEM((_NBUF, C, _EMB), jnp.int32),
            pltpu.VMEM((rows_per,), jnp.float32),
            pltpu.SemaphoreType.DMA((_NBUF + 1,)),
        ],
    )
    return run(h, t, r, ent_emb, relnorm)


# speedup vs baseline: 1.1103x; 1.0173x over previous
import functools

import jax
import jax.numpy as jnp
from jax import lax
from jax.experimental import pallas as pl
from jax.experimental.pallas import tpu as pltpu
from jax.experimental.pallas import tpu_sc as plsc

_EMB = 128
_NFV = _EMB // 16
_NBUF = 8

_GATHER_DNUMS = lax.GatherDimensionNumbers(
    offset_dims=(), collapsed_slice_dims=(0,), start_index_map=(0,))


def _lane_shuffle(x, idx):
    return lax.gather(x, idx[:, None], _GATHER_DNUMS, slice_sizes=(1,),
                      mode=lax.GatherScatterMode.PROMISE_IN_BOUNDS)


def _lane_sum(x, lanes):
    for sh in (8, 4, 2, 1):
        x = x + _lane_shuffle(x, jnp.bitwise_xor(lanes, sh))
    return x


def _sc_body(rows_per, C, h_hbm, t_hbm, r_hbm, ent_hbm, relnorm_hbm,
             out_hbm, h_idx, t_idx, r_idx, h_rows, t_rows, rn_rows,
             out_v, sem):
    nc = 2
    sid = lax.axis_index("s")
    wid = sid * nc + lax.axis_index("c")
    lanes = lax.iota(jnp.int32, 16)
    nchunks = rows_per // C
    wbase = wid * rows_per

    idx_cps = (
        pltpu.make_async_copy(h_hbm.at[pl.ds(wbase, rows_per)], h_idx,
                              sem.at[_NBUF]),
        pltpu.make_async_copy(t_hbm.at[pl.ds(wbase, rows_per)], t_idx,
                              sem.at[_NBUF]),
        pltpu.make_async_copy(r_hbm.at[pl.ds(wbase, rows_per)], r_idx,
                              sem.at[_NBUF]),
    )
    for cp in idx_cps:
        cp.start()
    for cp in idx_cps:
        cp.wait()

    def copies(c, b):
        off = c * C
        return (
            pltpu.make_async_copy(ent_hbm.at[h_idx.at[pl.ds(off, C)]],
                                  h_rows.at[b], sem.at[b]),
            pltpu.make_async_copy(ent_hbm.at[t_idx.at[pl.ds(off, C)]],
                                  t_rows.at[b], sem.at[b]),
            pltpu.make_async_copy(relnorm_hbm.at[r_idx.at[pl.ds(off, C)]],
                                  rn_rows.at[b], sem.at[b]),
        )

    def fire(c, b):
        for cp in copies(c, b):
            cp.start()

    def drain(c, b):
        for cp in copies(c, b):
            cp.wait()

    def compute(c, b):
        def score_row(i):
            def load2(ref, col16):
                w = ref[b, i, pl.ds(col16 * 16, 16)]
                lo = lax.bitcast_convert_type(
                    lax.shift_left(w, 16), jnp.float32)
                hi = lax.bitcast_convert_type(
                    jnp.bitwise_and(w, jnp.int32(-65536)), jnp.float32)
                return lo, hi

            dk = []
            nk = []
            accd = jnp.zeros((16,), jnp.float32)
            for k in range(_NFV // 2):
                n2 = load2(rn_rows, _NFV // 2 + k)
                for u in range(2):
                    kk = 2 * k + u
                    hv = h_rows[b, i, pl.ds(kk * 16, 16)]
                    tv = t_rows[b, i, pl.ds(kk * 16, 16)]
                    d = hv - tv
                    dk.append(d)
                    nk.append(n2[u])
                    accd = accd + d * n2[u]
            s = _lane_sum(accd, lanes)
            acc2 = jnp.zeros((16,), jnp.float32)
            for k in range(_NFV // 2):
                r2 = load2(rn_rows, k)
                for u in range(2):
                    e = dk[2 * k + u] + r2[u] - s * nk[2 * k + u]
                    acc2 = acc2 + jnp.abs(e)
            return _lane_sum(acc2, lanes)

        def group_body(g, _):
            def row_body(q, res):
                sc0 = score_row(g * 16 + q)
                return jnp.where(lanes == q, sc0, res)

            res = lax.fori_loop(0, 16, row_body,
                                jnp.zeros((16,), jnp.float32))
            out_v[pl.ds(c * C + g * 16, 16)] = res
            return 0

        lax.fori_loop(0, C // 16, group_body, 0)

    for b0 in range(_NBUF - 1):
        fire(b0, b0)

    def ring_body(c):
        b = lax.rem(c, _NBUF)

        @pl.when(c + _NBUF - 1 < nchunks)
        def _():
            fire(c + _NBUF - 1, lax.rem(c + _NBUF - 1, _NBUF))

        drain(c, b)
        compute(c, b)

    pl.loop(0, nchunks)(ring_body)
    pltpu.sync_copy(out_v, out_hbm.at[pl.ds(wbase, rows_per)])


def kernel(h, t, r, ent_emb, rel_emb, norm_emb):
    batch = h.shape[0]
    nw = 32
    rows_per = batch // nw
    C = 16
    def to_i32pairs(x):
        n = x.shape[0]
        xb = x.astype(jnp.bfloat16).reshape(n, x.shape[1] // 32, 2, 16)
        xb = jnp.transpose(xb, (0, 1, 3, 2))
        return lax.bitcast_convert_type(xb, jnp.int32).reshape(n, -1)

    relnorm = jnp.concatenate(
        [to_i32pairs(rel_emb), to_i32pairs(norm_emb)], axis=1)
    mesh = plsc.VectorSubcoreMesh(core_axis_name="c", subcore_axis_name="s")
    run = pl.kernel(
        functools.partial(_sc_body, rows_per, C),
        out_type=jax.ShapeDtypeStruct((batch,), jnp.float32),
        mesh=mesh,
        scratch_types=[
            pltpu.VMEM((rows_per,), jnp.int32),
            pltpu.VMEM((rows_per,), jnp.int32),
            pltpu.VMEM((rows_per,), jnp.int32),
            pltpu.VMEM((_NBUF, C, _EMB), jnp.float32),
            pltpu.VMEM((_NBUF, C, _EMB), jnp.float32),
            pltpu.VMEM((_NBUF, C, _EMB), jnp.int32),
            pltpu.VMEM((rows_per,), jnp.float32),
            pltpu.SemaphoreType.DMA((_NBUF + 1,)),
        ],
    )
    return run(h, t, r, ent_emb, relnorm)
